# baseline (speedup 1.0000x reference)
import functools

import numpy as np
import jax
import jax.numpy as jnp
from jax import lax
from jax.experimental import pallas as pl
from jax.experimental.pallas import tpu as pltpu
from jax.experimental.pallas import tpu_sc as plsc

B = 4
N = 900
I_TOT = 5440
H = 8
L = 4
P = 4
HEAD_DIM = 32
EMB = 256
QOUT = 384
NQ = B * N
ROWS_Q = 4 * H * L * P

NW = 32
Q_PER_W = -(-NQ // NW)

_LEVEL_W = (64.0, 32.0, 16.0, 8.0)
_LEVEL_START = (0, 4096, 5120, 5376)


def _xproj_body(img_ref, w_ref, b_ref, o_ref):
    acc = jax.lax.dot_general(
        img_ref[0], w_ref[...], (((1,), (0,)), ((), ())),
        preferred_element_type=jnp.float32)
    o_ref[...] = (acc + b_ref[...]).astype(jnp.bfloat16)[None]


def _xproj(img, w_t, bias2d):
    tile = 680
    return pl.pallas_call(
        _xproj_body,
        grid=(B, I_TOT // tile),
        in_specs=[
            pl.BlockSpec((1, tile, EMB), lambda b, i: (b, i, 0)),
            pl.BlockSpec((EMB, EMB), lambda b, i: (0, 0)),
            pl.BlockSpec((1, EMB), lambda b, i: (0, 0)),
        ],
        out_specs=pl.BlockSpec((1, tile, EMB), lambda b, i: (b, i, 0)),
        out_shape=jax.ShapeDtypeStruct((B, I_TOT, EMB), jnp.bfloat16),
    )(img, w_t, bias2d)


def _prep_body(q_ref, rp_ref, w_ref, b_ref, idx_ref, wt_ref):
    b = pl.program_id(0)
    q = jax.lax.dot_general(
        q_ref[0], w_ref[...], (((1,), (0,)), ((), ())),
        preferred_element_type=jnp.float32,
        precision=jax.lax.Precision.HIGHEST) + b_ref[...]
    xo = q[:, 0:128]
    yo = q[:, 128:256]
    lg = q[:, 256:384]

    m = jnp.max(lg, axis=1, keepdims=True)
    e = jnp.exp(lg - m)
    ri = lax.broadcasted_iota(jnp.int32, (128, 128), 0) // 16
    ci = lax.broadcasted_iota(jnp.int32, (128, 128), 1) // 16
    bd = jnp.where(ri == ci, 1.0, 0.0)
    s = jax.lax.dot_general(
        e, bd, (((1,), (0,)), ((), ())),
        preferred_element_type=jnp.float32,
        precision=jax.lax.Precision.HIGHEST)
    attn = e / s

    rx = rp_ref[0, :, 0:1]
    ry = rp_ref[0, :, 1:2]

    lane = lax.broadcasted_iota(jnp.int32, (N, 128), 1)
    li = (lane // 4) % 4
    h_idx = lane // 16
    wf = jnp.where(li == 0, _LEVEL_W[0],
                   jnp.where(li == 1, _LEVEL_W[1],
                             jnp.where(li == 2, _LEVEL_W[2], _LEVEL_W[3])))
    wi = jnp.where(li == 0, 64, jnp.where(li == 1, 32,
                                          jnp.where(li == 2, 16, 8)))
    start = jnp.where(li == 0, _LEVEL_START[0],
                      jnp.where(li == 1, _LEVEL_START[1],
                                jnp.where(li == 2, _LEVEL_START[2],
                                          _LEVEL_START[3])))

    ix = rx * wf + xo - 0.5
    iy = ry * wf + yo - 0.5
    x0 = jnp.floor(ix)
    y0 = jnp.floor(iy)
    wx1 = ix - x0
    wx0 = 1.0 - wx1
    wy1 = iy - y0
    wy0 = 1.0 - wy1

    base = b * (I_TOT * H)
    for cn, (cx, cy) in enumerate(((0, 0), (1, 0), (0, 1), (1, 1))):
        xf = x0 + cx
        yf = y0 + cy
        wxy = (wx1 if cx else wx0) * (wy1 if cy else wy0)
        valid = ((xf >= 0) & (xf <= wf - 1) & (yf >= 0) & (yf <= wf - 1))
        xc = jnp.clip(xf, 0, wf - 1).astype(jnp.int32)
        yc = jnp.clip(yf, 0, wf - 1).astype(jnp.int32)
        pix = start + yc * wi + xc
        row = base + pix * H + h_idx
        wgt = attn * wxy * jnp.where(valid, 1.0, 0.0)
        idx_ref[0, :, cn] = row
        wt_ref[0, :, cn] = wgt


def _prep(queries, rp, w_qpT, b_qp):
    return pl.pallas_call(
        _prep_body,
        grid=(B,),
        in_specs=[
            pl.BlockSpec((1, N, EMB), lambda b: (b, 0, 0)),
            pl.BlockSpec((1, N, 2), lambda b: (b, 0, 0)),
            pl.BlockSpec((EMB, QOUT), lambda b: (0, 0)),
            pl.BlockSpec((1, QOUT), lambda b: (0, 0)),
        ],
        out_specs=[
            pl.BlockSpec((1, N, 4, 128), lambda b: (b, 0, 0, 0)),
            pl.BlockSpec((1, N, 4, 128), lambda b: (b, 0, 0, 0)),
        ],
        out_shape=[
            jax.ShapeDtypeStruct((B, N, 4, 128), jnp.int32),
            jax.ShapeDtypeStruct((B, N, 4, 128), jnp.float32),
        ],
    )(queries, rp, w_qpT, b_qp)


def _sc_gather_combine(idxq, wtq, table):
    mesh = plsc.VectorSubcoreMesh(core_axis_name="c", subcore_axis_name="s")

    @functools.partial(
        pl.kernel,
        mesh=mesh,
        out_type=jax.ShapeDtypeStruct((NQ, EMB), jnp.float32),
        compiler_params=pltpu.CompilerParams(
            use_tc_tiling_on_sc=False, needs_layout_passes=False),
        scratch_types=[
            pltpu.VMEM((2, 4, 128), jnp.int32),
            pltpu.VMEM((2, 4, 128), jnp.float32),
            pltpu.VMEM((2, ROWS_Q, HEAD_DIM // 2), jnp.int32),
            pltpu.VMEM((2, EMB), jnp.float32),
            pltpu.SemaphoreType.DMA,
            pltpu.SemaphoreType.DMA,
            pltpu.SemaphoreType.DMA,
            pltpu.SemaphoreType.DMA,
            pltpu.SemaphoreType.DMA,
            pltpu.SemaphoreType.DMA,
        ],
    )
    def k(idx_h, wt_h, tab_h, out_h, idx_v, wt_v, rows_v, out_v,
          sf0, sf1, sg0, sg1, ss0, ss1):
        wid = lax.axis_index("s") * 2 + lax.axis_index("c")
        sf = (sf0, sf1)
        sg = (sg0, sg1)
        ss = (ss0, ss1)

        def fetch(t, p):
            @pl.when(t * NW + wid < NQ)
            def _():
                pltpu.async_copy(idx_h.at[t * NW + wid], idx_v.at[p], sf[p])
                pltpu.async_copy(wt_h.at[t * NW + wid], wt_v.at[p], sf[p])

        def wait_fetch(t, p):
            @pl.when(t * NW + wid < NQ)
            def _():
                pltpu.make_async_copy(idx_h.at[0], idx_v.at[p], sf[p]).wait()
                pltpu.make_async_copy(wt_h.at[0], wt_v.at[p], sf[p]).wait()

        def gathers(t, p):
            @pl.when(t * NW + wid < NQ)
            def _():
                for cn in range(4):
                    pltpu.async_copy(tab_h.at[idx_v.at[p, cn]],
                                     rows_v.at[p, pl.ds(cn * 128, 128)],
                                     sg[p])

        def wait_gathers(t, p):
            @pl.when(t * NW + wid < NQ)
            def _():
                pltpu.make_async_copy(tab_h.at[pl.ds(0, ROWS_Q)],
                                      rows_v.at[p], sg[p]).wait()

        def wait_store(t, p):
            @pl.when((t >= 0) & (t * NW + wid < NQ))
            def _():
                pltpu.make_async_copy(out_v.at[p], out_h.at[0], ss[p]).wait()

        def compute(t, p):
            qi = t * NW + wid

            @pl.when(qi < NQ)
            def _():
                def hbody(h, c2):
                    acc0 = jnp.zeros((16,), jnp.float32)
                    acc1 = jnp.zeros((16,), jnp.float32)
                    for cn in range(4):
                        wrow = wt_v[p, cn, pl.ds(h * 16, 16)]
                        for lp in range(16):
                            slot = cn * 128 + h * 16 + lp
                            wgt = wrow[lp]
                            w32 = plsc.bitcast(rows_v[p, slot, :],
                                               jnp.bfloat16)
                            ra, rb = plsc.unpack(
                                w32, format=plsc.PackFormat.INTERLEAVED)
                            acc0 = acc0 + wgt * ra
                            acc1 = acc1 + wgt * rb
                    out_v[p, pl.ds(h * 32, 16)] = acc0
                    out_v[p, pl.ds(h * 32 + 16, 16)] = acc1
                    return c2

                lax.fori_loop(0, H, hbody, 0)
                pltpu.async_copy(out_v.at[p], out_h.at[qi], ss[p])

        fetch(0, 0)
        wait_fetch(0, 0)
        gathers(0, 0)
        fetch(1, 1)

        def body(kk, carry):
            for u in range(2):
                t = 2 * kk + u
                p = u
                wait_gathers(t, p)
                wait_fetch(t + 1, 1 - p)
                gathers(t + 1, 1 - p)
                wait_store(t - 2, p)
                compute(t, p)
                fetch(t + 2, p)
            return carry

        lax.fori_loop(0, (Q_PER_W + 1) // 2, body, 0)
        wait_store(2 * ((Q_PER_W + 1) // 2) - 2, 0)

    return k(idxq, wtq, table)


def _outproj_body(h_ref, w_ref, b_ref, o_ref):
    acc = jax.lax.dot_general(
        h_ref[0], w_ref[...], (((1,), (0,)), ((), ())),
        preferred_element_type=jnp.float32,
        precision=jax.lax.Precision.HIGHEST)
    o_ref[...] = (acc + b_ref[...])[None]


def _outproj(heads, w_t, bias2d):
    return pl.pallas_call(
        _outproj_body,
        grid=(B,),
        in_specs=[
            pl.BlockSpec((1, N, EMB), lambda b: (b, 0, 0)),
            pl.BlockSpec((EMB, EMB), lambda b: (0, 0)),
            pl.BlockSpec((1, EMB), lambda b: (0, 0)),
        ],
        out_specs=pl.BlockSpec((1, N, EMB), lambda b: (b, 0, 0)),
        out_shape=jax.ShapeDtypeStruct((B, N, EMB), jnp.float32),
    )(heads, w_t, bias2d)


_HLP = np.arange(H * L * P)
_WQ_ORDER = np.concatenate([_HLP * 3 + c for c in range(3)])

_CH_PERM = np.concatenate(
    [np.concatenate([np.arange(0, 32, 2), np.arange(1, 32, 2)]) + 32 * h
     for h in range(H)])


def kernel(img, img_shapes, queries, reference_points,
           W_img, b_img, W_q, b_q, W_out, b_out):
    x = _xproj(img.astype(jnp.bfloat16), W_img.T.astype(jnp.bfloat16),
               b_img[None])
    table = jax.lax.bitcast_convert_type(
        x.reshape(B * I_TOT * H, HEAD_DIM // 2, 2), jnp.int32)
    w_qpT = jnp.take(W_q, _WQ_ORDER, axis=0).T
    b_qp = jnp.take(b_q, _WQ_ORDER)[None]
    idx4, wt4 = _prep(queries, reference_points, w_qpT, b_qp)
    idxq = idx4.reshape(NQ, 4, 128)
    wtq = wt4.reshape(NQ, 4, 128)
    heads = _sc_gather_combine(idxq, wtq, table)
    return _outproj(heads.reshape(B, N, EMB), W_out.T[_CH_PERM], b_out[None])


# speedup vs baseline: 17.2653x; 17.2653x over previous
import functools

import numpy as np
import jax
import jax.numpy as jnp
from jax import lax
from jax.experimental import pallas as pl
from jax.experimental.pallas import tpu as pltpu
from jax.experimental.pallas import tpu_sc as plsc

B = 4
N = 900
I_TOT = 5440
H = 8
L = 4
P = 4
HEAD_DIM = 32
EMB = 256
QOUT = 384
NQ = B * N
ROWS_Q = 4 * H * L * P

NW = 32
Q_PER_W = -(-NQ // NW)

_LEVEL_W = (64.0, 32.0, 16.0, 8.0)
_LEVEL_START = (0, 4096, 5120, 5376)


def _xproj_body(img_ref, w_ref, b_ref, o_ref):
    acc = jax.lax.dot_general(
        img_ref[0], w_ref[...], (((1,), (0,)), ((), ())),
        preferred_element_type=jnp.float32)
    acc = acc + b_ref[...]
    lo = lax.bitcast_convert_type(
        acc[:, 0:128].astype(jnp.bfloat16), jnp.int16).astype(jnp.int32)
    hi = lax.bitcast_convert_type(
        acc[:, 128:256].astype(jnp.bfloat16), jnp.int16).astype(jnp.int32)
    o_ref[...] = ((lo & 0xFFFF) | (hi << 16))[None]


def _xproj(img, w_t, bias2d):
    tile = 680
    return pl.pallas_call(
        _xproj_body,
        grid=(B, I_TOT // tile),
        in_specs=[
            pl.BlockSpec((1, tile, EMB), lambda b, i: (b, i, 0)),
            pl.BlockSpec((EMB, EMB), lambda b, i: (0, 0)),
            pl.BlockSpec((1, EMB), lambda b, i: (0, 0)),
        ],
        out_specs=pl.BlockSpec((1, tile, EMB // 2), lambda b, i: (b, i, 0)),
        out_shape=jax.ShapeDtypeStruct((B, I_TOT, EMB // 2), jnp.int32),
    )(img, w_t, bias2d)


def _prep_body(q_ref, rp_ref, w_ref, b_ref, idx_ref, wt_ref):
    b = pl.program_id(0)
    q = jax.lax.dot_general(
        q_ref[0], w_ref[...], (((1,), (0,)), ((), ())),
        preferred_element_type=jnp.float32,
        precision=jax.lax.Precision.HIGHEST) + b_ref[...]
    xo = q[:, 0:128]
    yo = q[:, 128:256]
    lg = q[:, 256:384]

    m = jnp.max(lg, axis=1, keepdims=True)
    e = jnp.exp(lg - m)
    ri = lax.broadcasted_iota(jnp.int32, (128, 128), 0) // 16
    ci = lax.broadcasted_iota(jnp.int32, (128, 128), 1) // 16
    bd = jnp.where(ri == ci, 1.0, 0.0)
    s = jax.lax.dot_general(
        e, bd, (((1,), (0,)), ((), ())),
        preferred_element_type=jnp.float32,
        precision=jax.lax.Precision.HIGHEST)
    attn = e / s

    rx = rp_ref[0, :, 0:1]
    ry = rp_ref[0, :, 1:2]

    lane = lax.broadcasted_iota(jnp.int32, (N, 128), 1)
    li = (lane // 4) % 4
    h_idx = lane // 16
    wf = jnp.where(li == 0, _LEVEL_W[0],
                   jnp.where(li == 1, _LEVEL_W[1],
                             jnp.where(li == 2, _LEVEL_W[2], _LEVEL_W[3])))
    wi = jnp.where(li == 0, 64, jnp.where(li == 1, 32,
                                          jnp.where(li == 2, 16, 8)))
    start = jnp.where(li == 0, _LEVEL_START[0],
                      jnp.where(li == 1, _LEVEL_START[1],
                                jnp.where(li == 2, _LEVEL_START[2],
                                          _LEVEL_START[3])))

    ix = rx * wf + xo - 0.5
    iy = ry * wf + yo - 0.5
    x0 = jnp.floor(ix)
    y0 = jnp.floor(iy)
    wx1 = ix - x0
    wx0 = 1.0 - wx1
    wy1 = iy - y0
    wy0 = 1.0 - wy1

    base = b * (I_TOT * H)
    for cn, (cx, cy) in enumerate(((0, 0), (1, 0), (0, 1), (1, 1))):
        xf = x0 + cx
        yf = y0 + cy
        wxy = (wx1 if cx else wx0) * (wy1 if cy else wy0)
        valid = ((xf >= 0) & (xf <= wf - 1) & (yf >= 0) & (yf <= wf - 1))
        xc = jnp.clip(xf, 0, wf - 1).astype(jnp.int32)
        yc = jnp.clip(yf, 0, wf - 1).astype(jnp.int32)
        pix = start + yc * wi + xc
        row = base + pix * H + h_idx
        wgt = attn * wxy * jnp.where(valid, 1.0, 0.0)
        idx_ref[0, :, cn] = row
        wt_ref[0, :, cn] = wgt


def _prep(queries, rp, w_qpT, b_qp):
    return pl.pallas_call(
        _prep_body,
        grid=(B,),
        in_specs=[
            pl.BlockSpec((1, N, EMB), lambda b: (b, 0, 0)),
            pl.BlockSpec((1, N, 2), lambda b: (b, 0, 0)),
            pl.BlockSpec((EMB, QOUT), lambda b: (0, 0)),
            pl.BlockSpec((1, QOUT), lambda b: (0, 0)),
        ],
        out_specs=[
            pl.BlockSpec((1, N, 4, 128), lambda b: (b, 0, 0, 0)),
            pl.BlockSpec((1, N, 4, 128), lambda b: (b, 0, 0, 0)),
        ],
        out_shape=[
            jax.ShapeDtypeStruct((B, N, 4, 128), jnp.int32),
            jax.ShapeDtypeStruct((B, N, 4, 128), jnp.float32),
        ],
    )(queries, rp, w_qpT, b_qp)


def _sc_gather_combine(idxq, wtq, table):
    mesh = plsc.VectorSubcoreMesh(core_axis_name="c", subcore_axis_name="s")

    @functools.partial(
        pl.kernel,
        mesh=mesh,
        out_type=jax.ShapeDtypeStruct((NQ, EMB), jnp.float32),
        compiler_params=pltpu.CompilerParams(
            use_tc_tiling_on_sc=False, needs_layout_passes=False),
        scratch_types=[
            pltpu.VMEM((2, 4, 128), jnp.int32),
            pltpu.VMEM((2, 4, 128), jnp.float32),
            pltpu.VMEM((2, ROWS_Q, HEAD_DIM // 2), jnp.int32),
            pltpu.VMEM((2, EMB), jnp.float32),
            pltpu.SemaphoreType.DMA,
            pltpu.SemaphoreType.DMA,
            pltpu.SemaphoreType.DMA,
            pltpu.SemaphoreType.DMA,
            pltpu.SemaphoreType.DMA,
            pltpu.SemaphoreType.DMA,
        ],
    )
    def k(idx_h, wt_h, tab_h, out_h, idx_v, wt_v, rows_v, out_v,
          sf0, sf1, sg0, sg1, ss0, ss1):
        wid = lax.axis_index("s") * 2 + lax.axis_index("c")
        sf = (sf0, sf1)
        sg = (sg0, sg1)
        ss = (ss0, ss1)

        def fetch(t, p):
            @pl.when(t * NW + wid < NQ)
            def _():
                pltpu.async_copy(idx_h.at[t * NW + wid], idx_v.at[p], sf[p])
                pltpu.async_copy(wt_h.at[t * NW + wid], wt_v.at[p], sf[p])

        def wait_fetch(t, p):
            @pl.when(t * NW + wid < NQ)
            def _():
                pltpu.make_async_copy(idx_h.at[0], idx_v.at[p], sf[p]).wait()
                pltpu.make_async_copy(wt_h.at[0], wt_v.at[p], sf[p]).wait()

        def gathers(t, p):
            @pl.when(t * NW + wid < NQ)
            def _():
                for cn in range(4):
                    pltpu.async_copy(tab_h.at[idx_v.at[p, cn]],
                                     rows_v.at[p, pl.ds(cn * 128, 128)],
                                     sg[p])

        def wait_gathers(t, p):
            @pl.when(t * NW + wid < NQ)
            def _():
                pltpu.make_async_copy(tab_h.at[pl.ds(0, ROWS_Q)],
                                      rows_v.at[p], sg[p]).wait()

        def wait_store(t, p):
            @pl.when((t >= 0) & (t * NW + wid < NQ))
            def _():
                pltpu.make_async_copy(out_v.at[p], out_h.at[0], ss[p]).wait()

        def compute(t, p):
            qi = t * NW + wid

            @pl.when(qi < NQ)
            def _():
                def hbody(h, c2):
                    acc0 = jnp.zeros((16,), jnp.float32)
                    acc1 = jnp.zeros((16,), jnp.float32)
                    for cn in range(4):
                        wrow = wt_v[p, cn, pl.ds(h * 16, 16)]
                        for lp in range(16):
                            slot = cn * 128 + h * 16 + lp
                            wgt = wrow[lp]
                            w32 = rows_v[p, slot, :]
                            ra = plsc.bitcast(w32 << 16, jnp.float32)
                            rb = plsc.bitcast(
                                w32 & jnp.int32(-65536), jnp.float32)
                            acc0 = acc0 + wgt * ra
                            acc1 = acc1 + wgt * rb
                    out_v[p, pl.ds(h * 32, 16)] = acc0
                    out_v[p, pl.ds(h * 32 + 16, 16)] = acc1
                    return c2

                lax.fori_loop(0, H, hbody, 0)
                pltpu.async_copy(out_v.at[p], out_h.at[qi], ss[p])

        fetch(0, 0)
        wait_fetch(0, 0)
        gathers(0, 0)
        fetch(1, 1)

        def body(kk, carry):
            for u in range(2):
                t = 2 * kk + u
                p = u
                wait_gathers(t, p)
                wait_fetch(t + 1, 1 - p)
                gathers(t + 1, 1 - p)
                wait_store(t - 2, p)
                compute(t, p)
                fetch(t + 2, p)
            return carry

        lax.fori_loop(0, (Q_PER_W + 1) // 2, body, 0)
        wait_store(2 * ((Q_PER_W + 1) // 2) - 2, 0)

    return k(idxq, wtq, table)


def _outproj_body(h_ref, w_ref, b_ref, o_ref):
    acc = jax.lax.dot_general(
        h_ref[0], w_ref[...], (((1,), (0,)), ((), ())),
        preferred_element_type=jnp.float32,
        precision=jax.lax.Precision.HIGHEST)
    o_ref[...] = (acc + b_ref[...])[None]


def _outproj(heads, w_t, bias2d):
    return pl.pallas_call(
        _outproj_body,
        grid=(B,),
        in_specs=[
            pl.BlockSpec((1, N, EMB), lambda b: (b, 0, 0)),
            pl.BlockSpec((EMB, EMB), lambda b: (0, 0)),
            pl.BlockSpec((1, EMB), lambda b: (0, 0)),
        ],
        out_specs=pl.BlockSpec((1, N, EMB), lambda b: (b, 0, 0)),
        out_shape=jax.ShapeDtypeStruct((B, N, EMB), jnp.float32),
    )(heads, w_t, bias2d)


_HLP = np.arange(H * L * P)
_WQ_ORDER = np.concatenate([_HLP * 3 + c for c in range(3)])

_XCH_PERM = np.concatenate([
    np.concatenate([np.arange(16) + 32 * h for h in range(H)]),
    np.concatenate([np.arange(16) + 32 * h + 16 for h in range(H)]),
])


def kernel(img, img_shapes, queries, reference_points,
           W_img, b_img, W_q, b_q, W_out, b_out):
    w_imgT = W_img.T[:, _XCH_PERM].astype(jnp.bfloat16)
    x = _xproj(img.astype(jnp.bfloat16), w_imgT,
               b_img[_XCH_PERM][None])
    table = x.reshape(B * I_TOT * H, HEAD_DIM // 2)
    w_qpT = jnp.take(W_q, _WQ_ORDER, axis=0).T
    b_qp = jnp.take(b_q, _WQ_ORDER)[None]
    idx4, wt4 = _prep(queries, reference_points, w_qpT, b_qp)
    idxq = idx4.reshape(NQ, 4, 128)
    wtq = wt4.reshape(NQ, 4, 128)
    heads = _sc_gather_combine(idxq, wtq, table)
    return _outproj(heads.reshape(B, N, EMB), W_out.T, b_out[None])


# speedup vs baseline: 17.2899x; 1.0014x over previous
import functools

import numpy as np
import jax
import jax.numpy as jnp
from jax import lax
from jax.experimental import pallas as pl
from jax.experimental.pallas import tpu as pltpu
from jax.experimental.pallas import tpu_sc as plsc

B = 4
N = 900
I_TOT = 5440
H = 8
L = 4
P = 4
HEAD_DIM = 32
EMB = 256
QOUT = 384
NQ = B * N
ROWS_Q = 4 * H * L * P

NW = 32
Q_PER_W = -(-NQ // NW)

_LEVEL_W = (64.0, 32.0, 16.0, 8.0)
_LEVEL_START = (0, 4096, 5120, 5376)


def _xproj_body(img_ref, w_ref, b_ref, o_ref):
    acc = jax.lax.dot_general(
        img_ref[0], w_ref[...], (((1,), (0,)), ((), ())),
        preferred_element_type=jnp.float32)
    acc = acc + b_ref[...]
    lo = lax.bitcast_convert_type(
        acc[:, 0:128].astype(jnp.bfloat16), jnp.int16).astype(jnp.int32)
    hi = lax.bitcast_convert_type(
        acc[:, 128:256].astype(jnp.bfloat16), jnp.int16).astype(jnp.int32)
    o_ref[...] = ((lo & 0xFFFF) | (hi << 16))[None]


def _xproj(img, w_t, bias2d):
    tile = 680
    return pl.pallas_call(
        _xproj_body,
        grid=(B, I_TOT // tile),
        in_specs=[
            pl.BlockSpec((1, tile, EMB), lambda b, i: (b, i, 0)),
            pl.BlockSpec((EMB, EMB), lambda b, i: (0, 0)),
            pl.BlockSpec((1, EMB), lambda b, i: (0, 0)),
        ],
        out_specs=pl.BlockSpec((1, tile, EMB // 2), lambda b, i: (b, i, 0)),
        out_shape=jax.ShapeDtypeStruct((B, I_TOT, EMB // 2), jnp.int32),
    )(img, w_t, bias2d)


def _prep_body(q_ref, rp_ref, w_ref, b_ref, idx_ref, wt_ref):
    b = pl.program_id(0)
    q = jax.lax.dot_general(
        q_ref[0], w_ref[...], (((1,), (0,)), ((), ())),
        preferred_element_type=jnp.float32,
        precision=jax.lax.Precision.HIGHEST) + b_ref[...]
    xo = q[:, 0:128]
    yo = q[:, 128:256]
    lg = q[:, 256:384]

    m = jnp.max(lg, axis=1, keepdims=True)
    e = jnp.exp(lg - m)
    ri = lax.broadcasted_iota(jnp.int32, (128, 128), 0) // 16
    ci = lax.broadcasted_iota(jnp.int32, (128, 128), 1) // 16
    bd = jnp.where(ri == ci, 1.0, 0.0)
    s = jax.lax.dot_general(
        e, bd, (((1,), (0,)), ((), ())),
        preferred_element_type=jnp.float32,
        precision=jax.lax.Precision.HIGHEST)
    attn = e / s

    rx = rp_ref[0, :, 0:1]
    ry = rp_ref[0, :, 1:2]

    lane = lax.broadcasted_iota(jnp.int32, (N, 128), 1)
    li = (lane // 4) % 4
    h_idx = lane // 16
    wf = jnp.where(li == 0, _LEVEL_W[0],
                   jnp.where(li == 1, _LEVEL_W[1],
                             jnp.where(li == 2, _LEVEL_W[2], _LEVEL_W[3])))
    wi = jnp.where(li == 0, 64, jnp.where(li == 1, 32,
                                          jnp.where(li == 2, 16, 8)))
    start = jnp.where(li == 0, _LEVEL_START[0],
                      jnp.where(li == 1, _LEVEL_START[1],
                                jnp.where(li == 2, _LEVEL_START[2],
                                          _LEVEL_START[3])))

    ix = rx * wf + xo - 0.5
    iy = ry * wf + yo - 0.5
    x0 = jnp.floor(ix)
    y0 = jnp.floor(iy)
    wx1 = ix - x0
    wx0 = 1.0 - wx1
    wy1 = iy - y0
    wy0 = 1.0 - wy1

    base = b * (I_TOT * H)
    for cn, (cx, cy) in enumerate(((0, 0), (1, 0), (0, 1), (1, 1))):
        xf = x0 + cx
        yf = y0 + cy
        wxy = (wx1 if cx else wx0) * (wy1 if cy else wy0)
        valid = ((xf >= 0) & (xf <= wf - 1) & (yf >= 0) & (yf <= wf - 1))
        xc = jnp.clip(xf, 0, wf - 1).astype(jnp.int32)
        yc = jnp.clip(yf, 0, wf - 1).astype(jnp.int32)
        pix = start + yc * wi + xc
        row = base + pix * H + h_idx
        wgt = attn * wxy * jnp.where(valid, 1.0, 0.0)
        idx_ref[0, :, cn] = row
        wt_ref[0, :, cn] = wgt


def _prep(queries, rp, w_qpT, b_qp):
    return pl.pallas_call(
        _prep_body,
        grid=(B,),
        in_specs=[
            pl.BlockSpec((1, N, EMB), lambda b: (b, 0, 0)),
            pl.BlockSpec((1, N, 2), lambda b: (b, 0, 0)),
            pl.BlockSpec((EMB, QOUT), lambda b: (0, 0)),
            pl.BlockSpec((1, QOUT), lambda b: (0, 0)),
        ],
        out_specs=[
            pl.BlockSpec((1, N, 4, 128), lambda b: (b, 0, 0, 0)),
            pl.BlockSpec((1, N, 4, 128), lambda b: (b, 0, 0, 0)),
        ],
        out_shape=[
            jax.ShapeDtypeStruct((B, N, 4, 128), jnp.int32),
            jax.ShapeDtypeStruct((B, N, 4, 128), jnp.float32),
        ],
    )(queries, rp, w_qpT, b_qp)


def _sc_gather_combine(idxq, wtq, table):
    mesh = plsc.VectorSubcoreMesh(core_axis_name="c", subcore_axis_name="s")

    @functools.partial(
        pl.kernel,
        mesh=mesh,
        out_type=jax.ShapeDtypeStruct((NQ, EMB), jnp.float32),
        compiler_params=pltpu.CompilerParams(
            use_tc_tiling_on_sc=False, needs_layout_passes=False),
        scratch_types=[
            pltpu.VMEM((2, 4, 128), jnp.int32),
            pltpu.VMEM((2, 4, 128), jnp.float32),
            pltpu.VMEM((2, ROWS_Q, HEAD_DIM // 2), jnp.int32),
            pltpu.VMEM((2, EMB), jnp.float32),
            pltpu.SemaphoreType.DMA,
            pltpu.SemaphoreType.DMA,
            pltpu.SemaphoreType.DMA,
            pltpu.SemaphoreType.DMA,
            pltpu.SemaphoreType.DMA,
            pltpu.SemaphoreType.DMA,
        ],
    )
    def k(idx_h, wt_h, tab_h, out_h, idx_v, wt_v, rows_v, out_v,
          sf0, sf1, sg0, sg1, ss0, ss1):
        wid = lax.axis_index("s") * 2 + lax.axis_index("c")
        sf = (sf0, sf1)
        sg = (sg0, sg1)
        ss = (ss0, ss1)

        def fetch(t, p):
            @pl.when(t * NW + wid < NQ)
            def _():
                pltpu.async_copy(idx_h.at[t * NW + wid], idx_v.at[p], sf[p])
                pltpu.async_copy(wt_h.at[t * NW + wid], wt_v.at[p], sf[p])

        def wait_fetch(t, p):
            @pl.when(t * NW + wid < NQ)
            def _():
                pltpu.make_async_copy(idx_h.at[0], idx_v.at[p], sf[p]).wait()
                pltpu.make_async_copy(wt_h.at[0], wt_v.at[p], sf[p]).wait()

        def gathers(t, p):
            @pl.when(t * NW + wid < NQ)
            def _():
                for cn in range(4):
                    pltpu.async_copy(tab_h.at[idx_v.at[p, cn]],
                                     rows_v.at[p, pl.ds(cn * 128, 128)],
                                     sg[p])

        def wait_gathers(t, p):
            @pl.when(t * NW + wid < NQ)
            def _():
                pltpu.make_async_copy(tab_h.at[pl.ds(0, ROWS_Q)],
                                      rows_v.at[p], sg[p]).wait()

        def wait_store(t, p):
            @pl.when((t >= 0) & (t * NW + wid < NQ))
            def _():
                pltpu.make_async_copy(out_v.at[p], out_h.at[0], ss[p]).wait()

        def compute(t, p):
            qi = t * NW + wid

            @pl.when(qi < NQ)
            def _():
                def hbody(h, c2):
                    acc0 = jnp.zeros((16,), jnp.float32)
                    acc1 = jnp.zeros((16,), jnp.float32)
                    for cn in range(4):
                        wrow = wt_v[p, cn, pl.ds(h * 16, 16)]
                        for lp in range(16):
                            slot = cn * 128 + h * 16 + lp
                            wgt = wrow[lp]
                            w32 = rows_v[p, slot, :]
                            ra = plsc.bitcast(w32 << 16, jnp.float32)
                            rb = plsc.bitcast(
                                w32 & jnp.int32(-65536), jnp.float32)
                            acc0 = acc0 + wgt * ra
                            acc1 = acc1 + wgt * rb
                    out_v[p, pl.ds(h * 32, 16)] = acc0
                    out_v[p, pl.ds(h * 32 + 16, 16)] = acc1
                    return c2

                lax.fori_loop(0, H, hbody, 0, unroll=4)
                pltpu.async_copy(out_v.at[p], out_h.at[qi], ss[p])

        fetch(0, 0)
        wait_fetch(0, 0)
        gathers(0, 0)
        fetch(1, 1)

        def body(kk, carry):
            for u in range(2):
                t = 2 * kk + u
                p = u
                wait_gathers(t, p)
                wait_fetch(t + 1, 1 - p)
                gathers(t + 1, 1 - p)
                wait_store(t - 2, p)
                compute(t, p)
                fetch(t + 2, p)
            return carry

        lax.fori_loop(0, (Q_PER_W + 1) // 2, body, 0)
        wait_store(2 * ((Q_PER_W + 1) // 2) - 2, 0)

    return k(idxq, wtq, table)


def _outproj_body(h_ref, w_ref, b_ref, o_ref):
    acc = jax.lax.dot_general(
        h_ref[0].astype(jnp.bfloat16), w_ref[...], (((1,), (0,)), ((), ())),
        preferred_element_type=jnp.float32)
    o_ref[...] = (acc + b_ref[...])[None]


def _outproj(heads, w_t, bias2d):
    return pl.pallas_call(
        _outproj_body,
        grid=(B,),
        in_specs=[
            pl.BlockSpec((1, N, EMB), lambda b: (b, 0, 0)),
            pl.BlockSpec((EMB, EMB), lambda b: (0, 0)),
            pl.BlockSpec((1, EMB), lambda b: (0, 0)),
        ],
        out_specs=pl.BlockSpec((1, N, EMB), lambda b: (b, 0, 0)),
        out_shape=jax.ShapeDtypeStruct((B, N, EMB), jnp.float32),
    )(heads, w_t, bias2d)


_HLP = np.arange(H * L * P)
_WQ_ORDER = np.concatenate([_HLP * 3 + c for c in range(3)])

_XCH_PERM = np.concatenate([
    np.concatenate([np.arange(16) + 32 * h for h in range(H)]),
    np.concatenate([np.arange(16) + 32 * h + 16 for h in range(H)]),
])


def kernel(img, img_shapes, queries, reference_points,
           W_img, b_img, W_q, b_q, W_out, b_out):
    w_imgT = W_img.T[:, _XCH_PERM].astype(jnp.bfloat16)
    x = _xproj(img.astype(jnp.bfloat16), w_imgT,
               b_img[_XCH_PERM][None])
    table = x.reshape(B * I_TOT * H, HEAD_DIM // 2)
    w_qpT = jnp.take(W_q, _WQ_ORDER, axis=0).T
    b_qp = jnp.take(b_q, _WQ_ORDER)[None]
    idx4, wt4 = _prep(queries, reference_points, w_qpT, b_qp)
    idxq = idx4.reshape(NQ, 4, 128)
    wtq = wt4.reshape(NQ, 4, 128)
    heads = _sc_gather_combine(idxq, wtq, table)
    return _outproj(heads.reshape(B, N, EMB),
                    W_out.T.astype(jnp.bfloat16), b_out[None])


# speedup vs baseline: 21.2294x; 1.2279x over previous
import functools

import numpy as np
import jax
import jax.numpy as jnp
from jax import lax
from jax.experimental import pallas as pl
from jax.experimental.pallas import tpu as pltpu
from jax.experimental.pallas import tpu_sc as plsc

B = 4
N = 900
I_TOT = 5440
H = 8
L = 4
P = 4
HEAD_DIM = 32
EMB = 256
QOUT = 384
NQ = B * N
ROWS_Q = 4 * H * L * P

NW = 32
Q_PER_W = -(-NQ // NW)

_LEVEL_W = (64.0, 32.0, 16.0, 8.0)
_LEVEL_START = (0, 4096, 5120, 5376)


def _xproj_body(img_ref, w_ref, b_ref, o_ref):
    acc = jax.lax.dot_general(
        img_ref[0].astype(jnp.bfloat16), w_ref[...], (((1,), (0,)), ((), ())),
        preferred_element_type=jnp.float32)
    acc = acc + b_ref[...]
    lo = lax.bitcast_convert_type(
        acc[:, 0:128].astype(jnp.bfloat16), jnp.int16).astype(jnp.int32)
    hi = lax.bitcast_convert_type(
        acc[:, 128:256].astype(jnp.bfloat16), jnp.int16).astype(jnp.int32)
    o_ref[...] = ((lo & 0xFFFF) | (hi << 16))[None]


def _xproj(img, w_t, bias2d):
    tile = 680
    return pl.pallas_call(
        _xproj_body,
        grid=(B, I_TOT // tile),
        in_specs=[
            pl.BlockSpec((1, tile, EMB), lambda b, i: (b, i, 0)),
            pl.BlockSpec((EMB, EMB), lambda b, i: (0, 0)),
            pl.BlockSpec((1, EMB), lambda b, i: (0, 0)),
        ],
        out_specs=pl.BlockSpec((1, tile, EMB // 2), lambda b, i: (b, i, 0)),
        out_shape=jax.ShapeDtypeStruct((B, I_TOT, EMB // 2), jnp.int32),
    )(img, w_t, bias2d)


def _prep_body(q_ref, rp_ref, w_ref, b_ref, idx_ref, wt_ref):
    b = pl.program_id(0)
    q = jax.lax.dot_general(
        q_ref[0], w_ref[...], (((1,), (0,)), ((), ())),
        preferred_element_type=jnp.float32) + b_ref[...]
    xo = q[:, 0:128]
    yo = q[:, 128:256]
    lg = q[:, 256:384]

    m = jnp.max(lg, axis=1, keepdims=True)
    e = jnp.exp(lg - m)
    ri = lax.broadcasted_iota(jnp.int32, (128, 128), 0) // 16
    ci = lax.broadcasted_iota(jnp.int32, (128, 128), 1) // 16
    bd = jnp.where(ri == ci, 1.0, 0.0)
    s = jax.lax.dot_general(
        e, bd, (((1,), (0,)), ((), ())),
        preferred_element_type=jnp.float32,
        precision=jax.lax.Precision.HIGHEST)
    attn = e / s

    rx = rp_ref[0, :, 0:1]
    ry = rp_ref[0, :, 1:2]

    lane = lax.broadcasted_iota(jnp.int32, (N, 128), 1)
    li = (lane // 4) % 4
    h_idx = lane // 16
    wf = jnp.where(li == 0, _LEVEL_W[0],
                   jnp.where(li == 1, _LEVEL_W[1],
                             jnp.where(li == 2, _LEVEL_W[2], _LEVEL_W[3])))
    wi = jnp.where(li == 0, 64, jnp.where(li == 1, 32,
                                          jnp.where(li == 2, 16, 8)))
    start = jnp.where(li == 0, _LEVEL_START[0],
                      jnp.where(li == 1, _LEVEL_START[1],
                                jnp.where(li == 2, _LEVEL_START[2],
                                          _LEVEL_START[3])))

    ix = rx * wf + xo - 0.5
    iy = ry * wf + yo - 0.5
    x0 = jnp.floor(ix)
    y0 = jnp.floor(iy)
    wx1 = ix - x0
    wx0 = 1.0 - wx1
    wy1 = iy - y0
    wy0 = 1.0 - wy1

    base = b * (I_TOT * H)
    for cn, (cx, cy) in enumerate(((0, 0), (1, 0), (0, 1), (1, 1))):
        xf = x0 + cx
        yf = y0 + cy
        wxy = (wx1 if cx else wx0) * (wy1 if cy else wy0)
        valid = ((xf >= 0) & (xf <= wf - 1) & (yf >= 0) & (yf <= wf - 1))
        xc = jnp.clip(xf, 0, wf - 1).astype(jnp.int32)
        yc = jnp.clip(yf, 0, wf - 1).astype(jnp.int32)
        pix = start + yc * wi + xc
        row = base + pix * H + h_idx
        wgt = attn * wxy * jnp.where(valid, 1.0, 0.0)
        idx_ref[0, :, cn] = row
        wt_ref[0, :, cn] = wgt


def _prep(queries, rp, w_qpT, b_qp):
    return pl.pallas_call(
        _prep_body,
        grid=(B,),
        in_specs=[
            pl.BlockSpec((1, N, EMB), lambda b: (b, 0, 0)),
            pl.BlockSpec((1, N, 2), lambda b: (b, 0, 0)),
            pl.BlockSpec((EMB, QOUT), lambda b: (0, 0)),
            pl.BlockSpec((1, QOUT), lambda b: (0, 0)),
        ],
        out_specs=[
            pl.BlockSpec((1, N, 4, 128), lambda b: (b, 0, 0, 0)),
            pl.BlockSpec((1, N, 4, 128), lambda b: (b, 0, 0, 0)),
        ],
        out_shape=[
            jax.ShapeDtypeStruct((B, N, 4, 128), jnp.int32),
            jax.ShapeDtypeStruct((B, N, 4, 128), jnp.float32),
        ],
    )(queries, rp, w_qpT, b_qp)


def _sc_gather_combine(idxq, wtq, table):
    mesh = plsc.VectorSubcoreMesh(core_axis_name="c", subcore_axis_name="s")

    @functools.partial(
        pl.kernel,
        mesh=mesh,
        out_type=jax.ShapeDtypeStruct((NQ, EMB), jnp.float32),
        compiler_params=pltpu.CompilerParams(
            use_tc_tiling_on_sc=False, needs_layout_passes=False),
        scratch_types=[
            pltpu.VMEM((2, 2, 4, 128), jnp.int32),
            pltpu.VMEM((2, 2, 4, 128), jnp.float32),
            pltpu.VMEM((2, 2 * ROWS_Q, HEAD_DIM // 2), jnp.int32),
            pltpu.VMEM((2, 2, EMB), jnp.float32),
            pltpu.SemaphoreType.DMA,
            pltpu.SemaphoreType.DMA,
            pltpu.SemaphoreType.DMA,
            pltpu.SemaphoreType.DMA,
            pltpu.SemaphoreType.DMA,
            pltpu.SemaphoreType.DMA,
        ],
    )
    def k(idx_h, wt_h, tab_h, out_h, idx_v, wt_v, rows_v, out_v,
          sf0, sf1, sg0, sg1, ss0, ss1):
        wid = lax.axis_index("s") * 2 + lax.axis_index("c")
        sf = (sf0, sf1)
        sg = (sg0, sg1)
        ss = (ss0, ss1)
        base = wid * (NQ // NW)

        def fetch(t, p):
            q0 = base + 2 * t
            pltpu.async_copy(idx_h.at[pl.ds(q0, 2)], idx_v.at[p], sf[p])
            pltpu.async_copy(wt_h.at[pl.ds(q0, 2)], wt_v.at[p], sf[p])

        def wait_fetch(p):
            pltpu.make_async_copy(idx_h.at[pl.ds(0, 2)], idx_v.at[p],
                                  sf[p]).wait()
            pltpu.make_async_copy(wt_h.at[pl.ds(0, 2)], wt_v.at[p],
                                  sf[p]).wait()

        def gathers(p):
            for qq in range(2):
                for cn in range(4):
                    pltpu.async_copy(
                        tab_h.at[idx_v.at[p, qq, cn]],
                        rows_v.at[p, pl.ds((qq * 4 + cn) * 128, 128)],
                        sg[p])

        def wait_gathers(p):
            pltpu.make_async_copy(tab_h.at[pl.ds(0, 2 * ROWS_Q)],
                                  rows_v.at[p], sg[p]).wait()

        def wait_store(p):
            pltpu.make_async_copy(out_v.at[p], out_h.at[pl.ds(0, 2)],
                                  ss[p]).wait()

        def combine_one(p, qq, wt_ref):
            def hbody(h, c2):
                acc0 = jnp.zeros((16,), jnp.float32)
                acc1 = jnp.zeros((16,), jnp.float32)
                for cn in range(4):
                    wrow = wt_ref[p, qq, cn, pl.ds(h * 16, 16)]
                    for lp in range(16):
                        slot = (qq * 4 + cn) * 128 + h * 16 + lp
                        wgt = wrow[lp]
                        w32 = rows_v[p, slot, :]
                        ra = plsc.bitcast(w32 << 16, jnp.float32)
                        rb = plsc.bitcast(w32 & jnp.int32(-65536),
                                          jnp.float32)
                        acc0 = acc0 + wgt * ra
                        acc1 = acc1 + wgt * rb
                out_v[p, qq, pl.ds(h * 32, 16)] = acc0
                out_v[p, qq, pl.ds(h * 32 + 16, 16)] = acc1
                return c2

            lax.fori_loop(0, H, hbody, 0)

        def compute(t, p):
            combine_one(p, 0, wt_v)
            combine_one(p, 1, wt_v)
            pltpu.async_copy(out_v.at[p], out_h.at[pl.ds(base + 2 * t, 2)],
                             ss[p])

        fetch(0, 0)
        wait_fetch(0)
        gathers(0)
        fetch(1, 1)

        def body(kk, carry):
            for u in range(2):
                t = 2 * kk + u
                p = u
                wait_gathers(p)
                wait_fetch(1 - p)
                gathers(1 - p)

                @pl.when(t >= 2)
                def _():
                    wait_store(p)

                compute(t, p)
                fetch(t + 2, p)
            return carry

        lax.fori_loop(0, 27, body, 0)
        wait_gathers(0)
        wait_fetch(1)
        gathers(1)
        wait_store(0)
        compute(54, 0)
        wait_gathers(1)
        wait_store(1)
        compute(55, 1)
        wait_store(0)
        wait_store(1)

        @pl.when(wid < NQ - NW * (NQ // NW))
        def _():
            qt = NW * (NQ // NW) + wid
            pltpu.async_copy(idx_h.at[pl.ds(qt, 1)],
                             idx_v.at[0, pl.ds(0, 1)], sf[0])
            pltpu.async_copy(wt_h.at[pl.ds(qt, 1)],
                             wt_v.at[0, pl.ds(0, 1)], sf[0])
            pltpu.make_async_copy(idx_h.at[pl.ds(0, 1)],
                                  idx_v.at[0, pl.ds(0, 1)], sf[0]).wait()
            pltpu.make_async_copy(wt_h.at[pl.ds(0, 1)],
                                  wt_v.at[0, pl.ds(0, 1)], sf[0]).wait()
            for cn in range(4):
                pltpu.async_copy(tab_h.at[idx_v.at[0, 0, cn]],
                                 rows_v.at[0, pl.ds(cn * 128, 128)], sg[0])
            pltpu.make_async_copy(tab_h.at[pl.ds(0, ROWS_Q)],
                                  rows_v.at[0, pl.ds(0, ROWS_Q)],
                                  sg[0]).wait()
            combine_one(0, 0, wt_v)
            pltpu.async_copy(out_v.at[0, pl.ds(0, 1)],
                             out_h.at[pl.ds(qt, 1)], ss[0])
            pltpu.make_async_copy(out_v.at[0, pl.ds(0, 1)],
                                  out_h.at[pl.ds(0, 1)], ss[0]).wait()

    return k(idxq, wtq, table)


def _outproj_body(h_ref, w_ref, b_ref, o_ref):
    acc = jax.lax.dot_general(
        h_ref[0].astype(jnp.bfloat16), w_ref[...], (((1,), (0,)), ((), ())),
        preferred_element_type=jnp.float32)
    o_ref[...] = (acc + b_ref[...])[None]


def _outproj(heads, w_t, bias2d):
    return pl.pallas_call(
        _outproj_body,
        grid=(B,),
        in_specs=[
            pl.BlockSpec((1, N, EMB), lambda b: (b, 0, 0)),
            pl.BlockSpec((EMB, EMB), lambda b: (0, 0)),
            pl.BlockSpec((1, EMB), lambda b: (0, 0)),
        ],
        out_specs=pl.BlockSpec((1, N, EMB), lambda b: (b, 0, 0)),
        out_shape=jax.ShapeDtypeStruct((B, N, EMB), jnp.float32),
    )(heads, w_t, bias2d)


_HLP = np.arange(H * L * P)
_WQ_ORDER = np.concatenate([_HLP * 3 + c for c in range(3)])

_XCH_PERM = np.concatenate([
    np.concatenate([np.arange(16) + 32 * h for h in range(H)]),
    np.concatenate([np.arange(16) + 32 * h + 16 for h in range(H)]),
])


def kernel(img, img_shapes, queries, reference_points,
           W_img, b_img, W_q, b_q, W_out, b_out):
    w_imgT = W_img.T[:, _XCH_PERM].astype(jnp.bfloat16)
    x = _xproj(img, w_imgT, b_img[_XCH_PERM][None])
    table = x.reshape(B * I_TOT * H, HEAD_DIM // 2)
    w_qpT = jnp.take(W_q, _WQ_ORDER, axis=0).T
    b_qp = jnp.take(b_q, _WQ_ORDER)[None]
    idx4, wt4 = _prep(queries, reference_points, w_qpT, b_qp)
    idxq = idx4.reshape(NQ, 4, 128)
    wtq = wt4.reshape(NQ, 4, 128)
    heads = _sc_gather_combine(idxq, wtq, table)
    return _outproj(heads.reshape(B, N, EMB),
                    W_out.T.astype(jnp.bfloat16), b_out[None])


# speedup vs baseline: 22.7399x; 1.0711x over previous
import functools

import numpy as np
import jax
import jax.numpy as jnp
from jax import lax
from jax.experimental import pallas as pl
from jax.experimental.pallas import tpu as pltpu
from jax.experimental.pallas import tpu_sc as plsc

B = 4
N = 900
I_TOT = 5440
H = 8
L = 4
P = 4
HEAD_DIM = 32
EMB = 256
QOUT = 384
NQ = B * N
ROWS_Q = 4 * H * L * P

NW = 32
Q_PER_W = -(-NQ // NW)

_LEVEL_W = (64.0, 32.0, 16.0, 8.0)
_LEVEL_START = (0, 4096, 5120, 5376)


def _xproj_body(img_ref, w_ref, b_ref, o_ref):
    acc = jax.lax.dot_general(
        img_ref[0].astype(jnp.bfloat16), w_ref[...], (((1,), (0,)), ((), ())),
        preferred_element_type=jnp.float32)
    acc = acc + b_ref[...]
    lo = lax.bitcast_convert_type(
        acc[:, 0:128].astype(jnp.bfloat16), jnp.int16).astype(jnp.int32)
    hi = lax.bitcast_convert_type(
        acc[:, 128:256].astype(jnp.bfloat16), jnp.int16).astype(jnp.int32)
    o_ref[...] = ((lo & 0xFFFF) | (hi << 16))[None]


def _xproj(img, w_t, bias2d):
    tile = 680
    return pl.pallas_call(
        _xproj_body,
        grid=(B, I_TOT // tile),
        in_specs=[
            pl.BlockSpec((1, tile, EMB), lambda b, i: (b, i, 0)),
            pl.BlockSpec((EMB, EMB), lambda b, i: (0, 0)),
            pl.BlockSpec((1, EMB), lambda b, i: (0, 0)),
        ],
        out_specs=pl.BlockSpec((1, tile, EMB // 2), lambda b, i: (b, i, 0)),
        out_shape=jax.ShapeDtypeStruct((B, I_TOT, EMB // 2), jnp.int32),
    )(img, w_t, bias2d)


def _prep_body(q_ref, rp_ref, w_ref, b_ref, idx_ref, wt_ref):
    b = pl.program_id(0)
    q = jax.lax.dot_general(
        q_ref[0], w_ref[...], (((1,), (0,)), ((), ())),
        preferred_element_type=jnp.float32) + b_ref[...]
    xo = q[:, 0:128]
    yo = q[:, 128:256]
    lg = q[:, 256:384]

    m = jnp.max(lg, axis=1, keepdims=True)
    e = jnp.exp(lg - m)
    ri = lax.broadcasted_iota(jnp.int32, (128, 128), 0) // 16
    ci = lax.broadcasted_iota(jnp.int32, (128, 128), 1) // 16
    bd = jnp.where(ri == ci, 1.0, 0.0)
    s = jax.lax.dot_general(
        e, bd, (((1,), (0,)), ((), ())),
        preferred_element_type=jnp.float32,
        precision=jax.lax.Precision.HIGHEST)
    attn = e / s

    rx = rp_ref[0, :, 0:1]
    ry = rp_ref[0, :, 1:2]

    lane = lax.broadcasted_iota(jnp.int32, (N, 128), 1)
    li = (lane // 4) % 4
    h_idx = lane // 16
    wf = jnp.where(li == 0, _LEVEL_W[0],
                   jnp.where(li == 1, _LEVEL_W[1],
                             jnp.where(li == 2, _LEVEL_W[2], _LEVEL_W[3])))
    wi = jnp.where(li == 0, 64, jnp.where(li == 1, 32,
                                          jnp.where(li == 2, 16, 8)))
    start = jnp.where(li == 0, _LEVEL_START[0],
                      jnp.where(li == 1, _LEVEL_START[1],
                                jnp.where(li == 2, _LEVEL_START[2],
                                          _LEVEL_START[3])))

    ix = rx * wf + xo - 0.5
    iy = ry * wf + yo - 0.5
    x0 = jnp.floor(ix)
    y0 = jnp.floor(iy)
    wx1 = ix - x0
    wx0 = 1.0 - wx1
    wy1 = iy - y0
    wy0 = 1.0 - wy1

    base = b * (I_TOT * H)
    for cn, (cx, cy) in enumerate(((0, 0), (1, 0), (0, 1), (1, 1))):
        xf = x0 + cx
        yf = y0 + cy
        wxy = (wx1 if cx else wx0) * (wy1 if cy else wy0)
        valid = ((xf >= 0) & (xf <= wf - 1) & (yf >= 0) & (yf <= wf - 1))
        xc = jnp.clip(xf, 0, wf - 1).astype(jnp.int32)
        yc = jnp.clip(yf, 0, wf - 1).astype(jnp.int32)
        pix = start + yc * wi + xc
        row = base + pix * H + h_idx
        wgt = attn * wxy * jnp.where(valid, 1.0, 0.0)
        idx_ref[0, :, cn] = row
        wt_ref[0, :, cn] = wgt


def _prep(queries, rp, w_qpT, b_qp):
    return pl.pallas_call(
        _prep_body,
        grid=(B,),
        in_specs=[
            pl.BlockSpec((1, N, EMB), lambda b: (b, 0, 0)),
            pl.BlockSpec((1, N, 2), lambda b: (b, 0, 0)),
            pl.BlockSpec((EMB, QOUT), lambda b: (0, 0)),
            pl.BlockSpec((1, QOUT), lambda b: (0, 0)),
        ],
        out_specs=[
            pl.BlockSpec((1, N, 4, 128), lambda b: (b, 0, 0, 0)),
            pl.BlockSpec((1, N, 4, 128), lambda b: (b, 0, 0, 0)),
        ],
        out_shape=[
            jax.ShapeDtypeStruct((B, N, 4, 128), jnp.int32),
            jax.ShapeDtypeStruct((B, N, 4, 128), jnp.float32),
        ],
    )(queries, rp, w_qpT, b_qp)


def _sc_gather_combine(idxq, wtq, table):
    mesh = plsc.VectorSubcoreMesh(core_axis_name="c", subcore_axis_name="s")

    @functools.partial(
        pl.kernel,
        mesh=mesh,
        out_type=jax.ShapeDtypeStruct((NQ, EMB), jnp.float32),
        compiler_params=pltpu.CompilerParams(
            use_tc_tiling_on_sc=False, needs_layout_passes=False),
        scratch_types=[
            pltpu.VMEM((2, 4, 4, 128), jnp.int32),
            pltpu.VMEM((2, 4, 4, 128), jnp.float32),
            pltpu.VMEM((2, 4 * ROWS_Q, HEAD_DIM // 2), jnp.int32),
            pltpu.VMEM((2, 4, EMB), jnp.float32),
            pltpu.SemaphoreType.DMA,
            pltpu.SemaphoreType.DMA,
            pltpu.SemaphoreType.DMA,
            pltpu.SemaphoreType.DMA,
            pltpu.SemaphoreType.DMA,
            pltpu.SemaphoreType.DMA,
        ],
    )
    def k(idx_h, wt_h, tab_h, out_h, idx_v, wt_v, rows_v, out_v,
          sf0, sf1, sg0, sg1, ss0, ss1):
        wid = lax.axis_index("s") * 2 + lax.axis_index("c")
        sf = (sf0, sf1)
        sg = (sg0, sg1)
        ss = (ss0, ss1)
        base = wid * (NQ // NW)

        def fetch(t, p):
            q0 = base + 4 * t
            pltpu.async_copy(idx_h.at[pl.ds(q0, 4)], idx_v.at[p], sf[p])
            pltpu.async_copy(wt_h.at[pl.ds(q0, 4)], wt_v.at[p], sf[p])

        def wait_fetch(p):
            pltpu.make_async_copy(idx_h.at[pl.ds(0, 4)], idx_v.at[p],
                                  sf[p]).wait()
            pltpu.make_async_copy(wt_h.at[pl.ds(0, 4)], wt_v.at[p],
                                  sf[p]).wait()

        def gathers(p):
            for qq in range(4):
                for cn in range(4):
                    pltpu.async_copy(
                        tab_h.at[idx_v.at[p, qq, cn]],
                        rows_v.at[p, pl.ds((qq * 4 + cn) * 128, 128)],
                        sg[p])

        def wait_gathers(p):
            pltpu.make_async_copy(tab_h.at[pl.ds(0, 4 * ROWS_Q)],
                                  rows_v.at[p], sg[p]).wait()

        def wait_store(p):
            pltpu.make_async_copy(out_v.at[p], out_h.at[pl.ds(0, 4)],
                                  ss[p]).wait()

        def combine_one(p, qq, wt_ref):
            def hbody(h, c2):
                acc0 = jnp.zeros((16,), jnp.float32)
                acc1 = jnp.zeros((16,), jnp.float32)
                for cn in range(4):
                    wrow = wt_ref[p, qq, cn, pl.ds(h * 16, 16)]
                    for lp in range(16):
                        slot = (qq * 4 + cn) * 128 + h * 16 + lp
                        wgt = wrow[lp]
                        w32 = rows_v[p, slot, :]
                        ra = plsc.bitcast(w32 << 16, jnp.float32)
                        rb = plsc.bitcast(w32 & jnp.int32(-65536),
                                          jnp.float32)
                        acc0 = acc0 + wgt * ra
                        acc1 = acc1 + wgt * rb
                out_v[p, qq, pl.ds(h * 32, 16)] = acc0
                out_v[p, qq, pl.ds(h * 32 + 16, 16)] = acc1
                return c2

            lax.fori_loop(0, H, hbody, 0)

        def compute(t, p):
            for qq in range(4):
                combine_one(p, qq, wt_v)
            pltpu.async_copy(out_v.at[p], out_h.at[pl.ds(base + 4 * t, 4)],
                             ss[p])

        fetch(0, 0)
        wait_fetch(0)
        gathers(0)
        fetch(1, 1)

        def body(kk, carry):
            for u in range(2):
                t = 2 * kk + u
                p = u
                wait_gathers(p)
                wait_fetch(1 - p)
                gathers(1 - p)

                @pl.when(t >= 2)
                def _():
                    wait_store(p)

                compute(t, p)
                fetch(t + 2, p)
            return carry

        lax.fori_loop(0, 13, body, 0)
        wait_gathers(0)
        wait_fetch(1)
        gathers(1)
        wait_store(0)
        compute(26, 0)
        wait_gathers(1)
        wait_store(1)
        compute(27, 1)
        wait_store(0)
        wait_store(1)

        @pl.when(wid < NQ - NW * (NQ // NW))
        def _():
            qt = NW * (NQ // NW) + wid
            pltpu.async_copy(idx_h.at[pl.ds(qt, 1)],
                             idx_v.at[0, pl.ds(0, 1)], sf[0])
            pltpu.async_copy(wt_h.at[pl.ds(qt, 1)],
                             wt_v.at[0, pl.ds(0, 1)], sf[0])
            pltpu.make_async_copy(idx_h.at[pl.ds(0, 1)],
                                  idx_v.at[0, pl.ds(0, 1)], sf[0]).wait()
            pltpu.make_async_copy(wt_h.at[pl.ds(0, 1)],
                                  wt_v.at[0, pl.ds(0, 1)], sf[0]).wait()
            for cn in range(4):
                pltpu.async_copy(tab_h.at[idx_v.at[0, 0, cn]],
                                 rows_v.at[0, pl.ds(cn * 128, 128)], sg[0])
            pltpu.make_async_copy(tab_h.at[pl.ds(0, ROWS_Q)],
                                  rows_v.at[0, pl.ds(0, ROWS_Q)],
                                  sg[0]).wait()
            combine_one(0, 0, wt_v)
            pltpu.async_copy(out_v.at[0, pl.ds(0, 1)],
                             out_h.at[pl.ds(qt, 1)], ss[0])
            pltpu.make_async_copy(out_v.at[0, pl.ds(0, 1)],
                                  out_h.at[pl.ds(0, 1)], ss[0]).wait()

    return k(idxq, wtq, table)


def _outproj_body(h_ref, w_ref, b_ref, o_ref):
    acc = jax.lax.dot_general(
        h_ref[0].astype(jnp.bfloat16), w_ref[...], (((1,), (0,)), ((), ())),
        preferred_element_type=jnp.float32)
    o_ref[...] = (acc + b_ref[...])[None]


def _outproj(heads, w_t, bias2d):
    return pl.pallas_call(
        _outproj_body,
        grid=(B,),
        in_specs=[
            pl.BlockSpec((1, N, EMB), lambda b: (b, 0, 0)),
            pl.BlockSpec((EMB, EMB), lambda b: (0, 0)),
            pl.BlockSpec((1, EMB), lambda b: (0, 0)),
        ],
        out_specs=pl.BlockSpec((1, N, EMB), lambda b: (b, 0, 0)),
        out_shape=jax.ShapeDtypeStruct((B, N, EMB), jnp.float32),
    )(heads, w_t, bias2d)


_HLP = np.arange(H * L * P)
_WQ_ORDER = np.concatenate([_HLP * 3 + c for c in range(3)])

_XCH_PERM = np.concatenate([
    np.concatenate([np.arange(16) + 32 * h for h in range(H)]),
    np.concatenate([np.arange(16) + 32 * h + 16 for h in range(H)]),
])


def kernel(img, img_shapes, queries, reference_points,
           W_img, b_img, W_q, b_q, W_out, b_out):
    w_imgT = W_img.T[:, _XCH_PERM].astype(jnp.bfloat16)
    x = _xproj(img, w_imgT, b_img[_XCH_PERM][None])
    table = x.reshape(B * I_TOT * H, HEAD_DIM // 2)
    w_qpT = jnp.take(W_q, _WQ_ORDER, axis=0).T
    b_qp = jnp.take(b_q, _WQ_ORDER)[None]
    idx4, wt4 = _prep(queries, reference_points, w_qpT, b_qp)
    idxq = idx4.reshape(NQ, 4, 128)
    wtq = wt4.reshape(NQ, 4, 128)
    heads = _sc_gather_combine(idxq, wtq, table)
    return _outproj(heads.reshape(B, N, EMB),
                    W_out.T.astype(jnp.bfloat16), b_out[None])
